# Initial kernel scaffold; baseline (speedup 1.0000x reference)
import jax
import jax.numpy as jnp
from jax.experimental import pallas as pl


def kernel(x, edge_index, W1, b1, W2, b2):
    raise NotImplementedError("write your pallas kernel here")


# speedup vs baseline: 23.9006x; 23.9006x over previous
import functools

import jax
import jax.numpy as jnp
from jax import lax
from jax.experimental import pallas as pl
from jax.experimental.pallas import tpu as pltpu
from jax.experimental.pallas import tpu_sc as plsc

NC = 2
NS = 16
CH = 128


def _mesh():
    return plsc.VectorSubcoreMesh(core_axis_name="c", subcore_axis_name="s")


def _make_edge_agg(n_rows, feat, cpw):
    rpt = n_rows // NS

    def body(table, srcp, dstp, zrows, out, src_v, dst_v, r0, r1, acc,
             sem0, sem1):
        c = lax.axis_index("c")
        s = lax.axis_index("s")
        wid = c * NS + s
        pltpu.sync_copy(zrows.at[pl.ds(s * rpt, rpt)],
                        acc.at[pl.ds(s * rpt, rpt)])
        pltpu.sync_copy(srcp.at[wid], src_v)
        pltpu.sync_copy(dstp.at[wid], dst_v)
        plsc.subcore_barrier()

        pltpu.async_copy(table.at[src_v.at[0]], r0, sem0)

        def step(i, carry):
            j = 2 * i
            pltpu.async_copy(table.at[src_v.at[j + 1]], r1, sem1)
            pltpu.make_async_copy(table.at[src_v.at[j]], r0, sem0).wait()
            pltpu.sync_copy(r0, acc.at[dst_v.at[j]], add=True)

            @pl.when(j + 2 < cpw)
            def _():
                pltpu.async_copy(table.at[src_v.at[j + 2]], r0, sem0)

            pltpu.make_async_copy(table.at[src_v.at[j + 1]], r1, sem1).wait()
            pltpu.sync_copy(r1, acc.at[dst_v.at[j + 1]], add=True)
            return carry

        lax.fori_loop(0, cpw // 2, step, 0)
        plsc.subcore_barrier()
        pltpu.sync_copy(acc.at[pl.ds(s * rpt, rpt)],
                        out.at[c, pl.ds(s * rpt, rpt)])

    return functools.partial(
        pl.kernel,
        out_type=jax.ShapeDtypeStruct((NC, n_rows, feat), jnp.float32),
        mesh=_mesh(),
        compiler_params=pltpu.CompilerParams(use_tc_tiling_on_sc=False),
        scratch_types=[
            pltpu.VMEM((cpw, CH), jnp.int32),
            pltpu.VMEM((cpw, CH), jnp.int32),
            pltpu.VMEM((CH, feat), jnp.float32),
            pltpu.VMEM((CH, feat), jnp.float32),
            pltpu.VMEM_SHARED((n_rows, feat), jnp.float32),
            pltpu.SemaphoreType.DMA,
            pltpu.SemaphoreType.DMA,
        ],
    )(body)


def _make_deg(n_rows, feat, cpw):
    rpt = n_rows // NS

    def body(dstp, ones_hbm, zrows, out, dst_v, ones_v, acc):
        c = lax.axis_index("c")
        s = lax.axis_index("s")
        wid = c * NS + s
        pltpu.sync_copy(zrows.at[pl.ds(s * rpt, rpt)],
                        acc.at[pl.ds(s * rpt, rpt)])
        pltpu.sync_copy(dstp.at[wid], dst_v)
        pltpu.sync_copy(ones_hbm, ones_v)
        plsc.subcore_barrier()

        def step(i, carry):
            pltpu.sync_copy(ones_v, acc.at[dst_v.at[i]], add=True)
            return carry

        lax.fori_loop(0, cpw, step, 0)
        plsc.subcore_barrier()
        pltpu.sync_copy(acc.at[pl.ds(s * rpt, rpt)],
                        out.at[c, pl.ds(s * rpt, rpt)])

    return functools.partial(
        pl.kernel,
        out_type=jax.ShapeDtypeStruct((NC, n_rows, feat), jnp.float32),
        mesh=_mesh(),
        compiler_params=pltpu.CompilerParams(use_tc_tiling_on_sc=False),
        scratch_types=[
            pltpu.VMEM((cpw, CH), jnp.int32),
            pltpu.VMEM((CH, feat), jnp.float32),
            pltpu.VMEM_SHARED((n_rows, feat), jnp.float32),
        ],
    )(body)


def _dense1(x_ref, w1_ref, d0_ref, d1_ref, hs_ref, dinv_ref):
    deg = d0_ref[:, 0:1] + d1_ref[:, 0:1] + 1.0
    dinv = lax.rsqrt(deg)
    h = jnp.dot(x_ref[...], w1_ref[...], preferred_element_type=jnp.float32)
    hs_ref[...] = h * dinv
    dinv_ref[...] = dinv


def _dense2(p0_ref, p1_ref, hs1_ref, dinv_ref, b1_ref, w2_ref, hs2_ref):
    dinv = dinv_ref[...]
    out1 = dinv * (p0_ref[...] + p1_ref[...] + hs1_ref[...]) + b1_ref[...]
    h = jnp.maximum(out1, 0.0)
    z = jnp.dot(h, w2_ref[...], preferred_element_type=jnp.float32)
    hs2_ref[...] = z * dinv


def _dense3(p0_ref, p1_ref, hs2_ref, dinv_ref, b2_ref, out_ref):
    dinv = dinv_ref[...]
    logits = dinv * (p0_ref[...] + p1_ref[...] + hs2_ref[...]) + b2_ref[...]
    ncls = 10
    mask = lax.broadcasted_iota(jnp.int32, logits.shape, 1) < ncls
    m = jnp.max(jnp.where(mask, logits, -1e30), axis=1, keepdims=True)
    e = jnp.where(mask, jnp.exp(logits - m), 0.0)
    lse = m + jnp.log(jnp.sum(e, axis=1, keepdims=True))
    out_ref[...] = logits - lse


def _tc_call(fn, out_shapes, *args):
    return pl.pallas_call(
        fn,
        out_shape=[jax.ShapeDtypeStruct(s, jnp.float32) for s in out_shapes],
    )(*args)


def kernel(x, edge_index, W1, b1, W2, b2):
    n, _ = x.shape
    h_dim = W1.shape[1]
    e = edge_index.shape[1]
    nw = NC * NS

    f2 = 16
    cpw = -(-e // (nw * CH))
    cpw += cpw % 2
    epw = cpw * CH
    n_rows = -(-(n + 1) // (NS * 8)) * (NS * 8)

    src = edge_index[0]
    dst = edge_index[1]
    srcp = jnp.pad(src, (0, epw * nw - e)).reshape(nw, cpw, CH)
    dstp = jnp.pad(dst, (0, epw * nw - e),
                   constant_values=n).reshape(nw, cpw, CH)

    zrows64 = jnp.zeros((n_rows, h_dim), jnp.float32)
    zrows16 = jnp.zeros((n_rows, f2), jnp.float32)
    ones16 = jnp.ones((CH, f2), jnp.float32)

    degp = _make_deg(n_rows, f2, cpw)(dstp, ones16, zrows16)

    hs1, dinv = _tc_call(
        _dense1, [(n, h_dim), (n, 1)],
        x, W1, degp[0, :n], degp[1, :n])

    aggp1 = _make_edge_agg(n_rows, h_dim, cpw)(hs1, srcp, dstp, zrows64)

    w2p = jnp.pad(W2, ((0, 0), (0, f2 - W2.shape[1])))
    (hs2,) = _tc_call(
        _dense2, [(n, f2)],
        aggp1[0, :n], aggp1[1, :n], hs1, dinv, b1.reshape(1, h_dim), w2p)

    aggp2 = _make_edge_agg(n_rows, f2, cpw)(hs2, srcp, dstp, zrows16)

    b2p = jnp.pad(b2, (0, f2 - b2.shape[0])).reshape(1, f2)
    (outp,) = _tc_call(
        _dense3, [(n, f2)],
        aggp2[0, :n], aggp2[1, :n], hs2, dinv, b2p)
    return outp[:, :W2.shape[1]]
